# baseline (speedup 1.0000x reference)
import functools

import jax
import jax.numpy as jnp
from jax import lax
from jax.experimental import pallas as pl
from jax.experimental.pallas import tpu as pltpu
from jax.experimental.pallas import tpu_sc as plsc

N = 10000
E = 320000
D = 128
G = 64
NC = 2
NS = 16
NW = NC * NS
CH = 128
CTP = 2688
EPAD = CTP * CH
EPTD = EPAD // NW
RPTD = EPTD // CH
NPAD = 10240
HR = 80
HC = 128
ZR = NPAD // NS
NB = 64
BW = NPAD // NB
CAP = 384
CHB = 64
DUMMY = N


def _elu(v):
    return jnp.where(v > 0, v, jnp.exp(v) - 1.0)


def _dinv_col(degp):
    deg = degp[0] + degp[1] + 1.0
    return lax.rsqrt(deg)


_MESH = dict(mesh=plsc.VectorSubcoreMesh(core_axis_name="c",
                                         subcore_axis_name="s"))


def _deg_body(dst_hbm, z_hbm, out_hbm, dstv, hist, rowidx, degacc):
    c = lax.axis_index("c")
    s = lax.axis_index("s")
    w = c * NS + s
    pltpu.sync_copy(z_hbm.at[pl.ds(0, HR)], hist)
    pltpu.sync_copy(dst_hbm.at[w], dstv)
    for i in range(HR // 16):
        rowidx[pl.ds(i * 16, 16)] = lax.iota(jnp.int32, 16) + (16 * i)

    @pl.when(s == 0)
    def _():
        pltpu.sync_copy(z_hbm.at[pl.ds(0, HR)], degacc)

    plsc.subcore_barrier()

    ones = jnp.ones((16,), jnp.float32)

    def step(i, carry):
        idx = dstv[i >> 3, pl.ds((i & 7) * 16, 16)]
        plsc.addupdate_scatter(hist, [idx >> 7, idx & 127], ones)
        return carry

    lax.fori_loop(0, EPTD // 16, step, 0)

    plsc.subcore_barrier()
    pltpu.sync_copy(hist, degacc.at[rowidx], add=True)
    plsc.subcore_barrier()

    @pl.when(s == 0)
    def _():
        pltpu.sync_copy(degacc, out_hbm.at[c])


_deg_call = functools.partial(
    pl.kernel,
    out_type=jax.ShapeDtypeStruct((NC, HR, HC), jnp.float32),
    scratch_types=[
        pltpu.VMEM((RPTD, CH), jnp.int32),
        pltpu.VMEM((HR, HC), jnp.float32),
        pltpu.VMEM((HR,), jnp.int32),
        pltpu.VMEM_SHARED((HR, HC), jnp.float32),
    ],
    compiler_params=pltpu.CompilerParams(needs_layout_passes=False),
    **_MESH,
)(_deg_body)


def _bkt_body(src_hbm, dst_hbm, lists_hbm, cnts_hbm, srcv, dstv, stg, cnt):
    c = lax.axis_index("c")
    s = lax.axis_index("s")
    w = c * NS + s
    pltpu.sync_copy(src_hbm.at[w], srcv)
    pltpu.sync_copy(dst_hbm.at[w], dstv)

    dm = jnp.full((16,), DUMMY, jnp.int32)

    def fill(r, carry):
        for q in range(CAP // 16):
            stg[r, pl.ds(q * 16, 16)] = dm
        return carry

    lax.fori_loop(0, NB, fill, 0)
    zi = jnp.zeros((16,), jnp.int32)
    for r in range(NB // 16):
        cnt[pl.ds(r * 16, 16)] = zi

    def place(i, carry):
        sv = srcv[i >> 3, pl.ds((i & 7) * 16, 16)]
        dv = dstv[i >> 3, pl.ds((i & 7) * 16, 16)]
        b16 = ((sv >> 5) * 52429) >> 18
        rank, lastm = plsc.scan_count(b16)
        base = plsc.load_gather(cnt, [b16])
        pos = jnp.minimum(base + rank - 1, CAP - 1)
        rec = ((sv - b16 * BW) << 14) | dv
        plsc.store_scatter(stg, [b16, pos], rec)
        plsc.store_scatter(cnt, [b16], jnp.minimum(base + rank, CAP),
                           mask=lastm)
        return carry

    lax.fori_loop(0, EPTD // 16, place, 0)

    pltpu.sync_copy(stg, lists_hbm.at[w])
    pltpu.sync_copy(cnt, cnts_hbm.at[w])


_bkt_call = functools.partial(
    pl.kernel,
    out_type=[jax.ShapeDtypeStruct((NW, NB, CAP), jnp.int32),
              jax.ShapeDtypeStruct((NW, NB), jnp.int32)],
    scratch_types=[
        pltpu.VMEM((RPTD, CH), jnp.int32),
        pltpu.VMEM((RPTD, CH), jnp.int32),
        pltpu.VMEM((NB, CAP), jnp.int32),
        pltpu.VMEM((NB,), jnp.int32),
    ],
    compiler_params=pltpu.CompilerParams(needs_layout_passes=False),
    **_MESH,
)(_bkt_body)


def _edge_body(g_hbm, lists_hbm, cntsT_hbm, z_hbm, out_hbm,
               tbl, stage, recs, crow, dstb, lsb, gsems, ssems, acc):
    c = lax.axis_index("c")
    s = lax.axis_index("s")
    w = c * NS + s
    pltpu.sync_copy(z_hbm, acc.at[pl.ds(s * ZR, ZR)])
    plsc.subcore_barrier()

    for r in range(NB // NW):
        b = w + NW * r
        pltpu.sync_copy(g_hbm.at[pl.ds(b * BW, BW)], tbl.at[pl.ds(s * BW, BW)])
        pltpu.sync_copy(cntsT_hbm.at[b], crow)

        def outer(w2, carry):
            pltpu.sync_copy(lists_hbm.at[w2, b], recs)
            cv = plsc.load_gather(crow, [jnp.full((16,), w2, jnp.int32)])
            nph = (cv[0] + (2 * CHB - 1)) >> 7

            def pair(i, carry2):
                for t in range(2):
                    base = (2 * i + t) * CHB
                    for grp in range(CHB // 16):
                        rec16 = recs[pl.ds(base + grp * 16, 16)]
                        dstb[t, pl.ds(grp * 16, 16)] = rec16 & 16383
                        lsb[t, pl.ds(grp * 16, 16)] = (rec16 >> 14) + s * BW
                    pltpu.async_copy(tbl.at[lsb.at[t]], stage.at[t], gsems[t])
                for t in range(2):
                    pltpu.make_async_copy(tbl.at[lsb.at[t]], stage.at[t],
                                          gsems[t]).wait()
                    pltpu.async_copy(stage.at[t], acc.at[dstb.at[t]],
                                     ssems[t], add=True)
                for t in range(2):
                    pltpu.make_async_copy(stage.at[t], acc.at[dstb.at[t]],
                                          ssems[t]).wait()
                return carry2

            lax.fori_loop(0, nph, pair, 0)
            return carry

        lax.fori_loop(0, NW, outer, 0)

    plsc.subcore_barrier()

    @pl.when(s == 0)
    def _():
        pltpu.sync_copy(acc, out_hbm.at[c])


_edge_call = functools.partial(
    pl.kernel,
    out_type=jax.ShapeDtypeStruct((NC, NPAD, D), jnp.float32),
    scratch_types=[
        pltpu.VMEM_SHARED((NS * BW, D), jnp.float32),
        pltpu.VMEM((2, CHB, D), jnp.float32),
        pltpu.VMEM((CAP,), jnp.int32),
        pltpu.VMEM((NW,), jnp.int32),
        pltpu.VMEM((2, CHB), jnp.int32),
        pltpu.VMEM((2, CHB), jnp.int32),
        [pltpu.SemaphoreType.DMA for _ in range(2)],
        [pltpu.SemaphoreType.DMA for _ in range(2)],
        pltpu.VMEM_SHARED((NPAD, D), jnp.float32),
    ],
    compiler_params=pltpu.CompilerParams(needs_layout_passes=False),
    **_MESH,
)(_edge_body)


def _tc_mm_body(x_ref, w_ref, o_ref):
    o_ref[...] = jnp.dot(x_ref[...], w_ref[...],
                         preferred_element_type=jnp.float32)


def _tc_scale_body(h0_ref, degp_ref, o_ref):
    o_ref[...] = h0_ref[...] * _dinv_col(degp_ref[...])


def _tc_combine_body(aggp_ref, g1_ref, degp_ref, b_ref, w2_ref, o_ref):
    dinv = _dinv_col(degp_ref[...])
    tot = aggp_ref[0] + aggp_ref[1] + g1_ref[...]
    h = _elu(dinv * tot + b_ref[...])
    g2 = jnp.dot(h, w2_ref[...], preferred_element_type=jnp.float32) * dinv
    row = lax.broadcasted_iota(jnp.int32, (NPAD, 1), 0)
    o_ref[...] = jnp.where(row < N, g2, 0.0)


def _tc_pool_body(aggp_ref, g2_ref, degp_ref, b_ref, batch_ref, o_ref):
    dinv = _dinv_col(degp_ref[...])
    tot = aggp_ref[0] + aggp_ref[1] + g2_ref[...]
    out2 = _elu(dinv * tot + b_ref[...])
    bcol = batch_ref[...]
    onehot = (bcol == lax.broadcasted_iota(jnp.int32, (1, G), 1)
              ).astype(jnp.float32)
    sums = lax.dot_general(onehot, out2, (((0,), (0,)), ((), ())),
                           preferred_element_type=jnp.float32)
    cnt = jnp.sum(onehot, axis=0).reshape(G, 1)
    o_ref[...] = sums / jnp.maximum(cnt, 1.0)


def _tc(body, out_shape):
    return pl.pallas_call(body, out_shape=out_shape)


def kernel(x, edge_index, batch, W1, b1, Wmu, bmu, Wsig, bsig):
    f32 = jnp.float32
    x_pad = jnp.pad(x, ((0, NPAD - N), (0, 0)))
    src_d = jnp.pad(edge_index[0], (0, EPAD - E),
                    constant_values=N).reshape(NW, RPTD, CH)
    dst_d = jnp.pad(edge_index[1], (0, EPAD - E),
                    constant_values=N).reshape(NW, RPTD, CH)
    batch_col = jnp.pad(batch, (0, NPAD - N), constant_values=G).reshape(NPAD, 1)
    zeros = jnp.zeros((ZR, D), f32)
    W2 = jnp.concatenate([Wmu, Wsig], axis=1)
    b2 = jnp.concatenate([bmu, bsig]).reshape(1, D)
    b1r = b1.reshape(1, D)

    degp = _deg_call(dst_d, zeros)
    degp = degp.reshape(NC, HR * HC, 1)

    lists, cnts = _bkt_call(src_d, dst_d)
    cntsT = cnts.T

    h0 = _tc(_tc_mm_body, jax.ShapeDtypeStruct((NPAD, D), f32))(x_pad, W1)
    g1 = _tc(_tc_scale_body, jax.ShapeDtypeStruct((NPAD, D), f32))(h0, degp)

    agg1 = _edge_call(g1, lists, cntsT, zeros)

    g2 = _tc(_tc_combine_body, jax.ShapeDtypeStruct((NPAD, D), f32))(
        agg1, g1, degp, b1r, W2)

    agg2 = _edge_call(g2, lists, cntsT, zeros)

    z = _tc(_tc_pool_body, jax.ShapeDtypeStruct((G, D), f32))(
        agg2, g2, degp, b2, batch_col)

    return (z[:, : D // 2], z[:, D // 2:])


# speedup vs baseline: 1.1964x; 1.1964x over previous
import functools

import jax
import jax.numpy as jnp
from jax import lax
from jax.experimental import pallas as pl
from jax.experimental.pallas import tpu as pltpu
from jax.experimental.pallas import tpu_sc as plsc

N = 10000
E = 320000
D = 128
G = 64
NC = 2
NS = 16
NW = NC * NS
CH = 128
CTP = 2688
EPAD = CTP * CH
EPTD = EPAD // NW
RPTD = EPTD // CH
NPAD = 10240
HR = 80
HC = 128
ZR = NPAD // NS
NB = 64
BW = NPAD // NB
CAP = 384
CHB = 64
DUMMY = N


def _elu(v):
    return jnp.where(v > 0, v, jnp.exp(v) - 1.0)


def _dinv_col(degp):
    deg = degp[0] + degp[1] + 1.0
    return lax.rsqrt(deg)


_MESH = dict(mesh=plsc.VectorSubcoreMesh(core_axis_name="c",
                                         subcore_axis_name="s"))


def _bkt_body(src_hbm, dst_hbm, z_hbm, deg_hbm, lists_hbm, cnts_hbm,
              srcv, dstv, stg, cnt, hist, rowidx, degacc):
    c = lax.axis_index("c")
    s = lax.axis_index("s")
    w = c * NS + s
    pltpu.sync_copy(src_hbm.at[w], srcv)
    pltpu.sync_copy(dst_hbm.at[w], dstv)
    pltpu.sync_copy(z_hbm.at[pl.ds(0, HR)], hist)
    for i in range(HR // 16):
        rowidx[pl.ds(i * 16, 16)] = lax.iota(jnp.int32, 16) + (16 * i)

    @pl.when(s == 0)
    def _():
        pltpu.sync_copy(z_hbm.at[pl.ds(0, HR)], degacc)

    dm = jnp.full((16,), DUMMY, jnp.int32)

    def fill(r, carry):
        for q in range(CAP // 16):
            stg[r, pl.ds(q * 16, 16)] = dm
        return carry

    lax.fori_loop(0, NB, fill, 0)
    zi = jnp.zeros((16,), jnp.int32)
    for r in range(NB // 16):
        cnt[pl.ds(r * 16, 16)] = zi

    plsc.subcore_barrier()

    ones = jnp.ones((16,), jnp.float32)

    def step(i, carry):
        dv = dstv[i >> 3, pl.ds((i & 7) * 16, 16)]
        plsc.addupdate_scatter(hist, [dv >> 7, dv & 127], ones)
        sv = srcv[i >> 3, pl.ds((i & 7) * 16, 16)]
        b16 = ((sv >> 5) * 52429) >> 18
        rank, lastm = plsc.scan_count(b16)
        base = plsc.load_gather(cnt, [b16])
        pos = jnp.minimum(base + rank - 1, CAP - 1)
        rec = ((sv - b16 * BW) << 14) | dv
        plsc.store_scatter(stg, [b16, pos], rec)
        plsc.store_scatter(cnt, [b16], jnp.minimum(base + rank, CAP),
                           mask=lastm)
        return carry

    lax.fori_loop(0, EPTD // 16, step, 0)

    pltpu.sync_copy(stg, lists_hbm.at[w])
    pltpu.sync_copy(cnt, cnts_hbm.at[w])
    plsc.subcore_barrier()
    pltpu.sync_copy(hist, degacc.at[rowidx], add=True)
    plsc.subcore_barrier()

    @pl.when(s == 0)
    def _():
        pltpu.sync_copy(degacc, deg_hbm.at[c])


_bkt_call = functools.partial(
    pl.kernel,
    out_type=[jax.ShapeDtypeStruct((NC, HR, HC), jnp.float32),
              jax.ShapeDtypeStruct((NW, NB, CAP), jnp.int32),
              jax.ShapeDtypeStruct((NW, NB), jnp.int32)],
    scratch_types=[
        pltpu.VMEM((RPTD, CH), jnp.int32),
        pltpu.VMEM((RPTD, CH), jnp.int32),
        pltpu.VMEM((NB, CAP), jnp.int32),
        pltpu.VMEM((NB,), jnp.int32),
        pltpu.VMEM((HR, HC), jnp.float32),
        pltpu.VMEM((HR,), jnp.int32),
        pltpu.VMEM_SHARED((HR, HC), jnp.float32),
    ],
    compiler_params=pltpu.CompilerParams(needs_layout_passes=False),
    **_MESH,
)(_bkt_body)


def _edge_body(g_hbm, lists_hbm, cntsT_hbm, z_hbm, out_hbm,
               tbl, stage, recs, crow, dstb, lsb, gsems, ssems, acc):
    c = lax.axis_index("c")
    s = lax.axis_index("s")
    w = c * NS + s
    pltpu.sync_copy(z_hbm, acc.at[pl.ds(s * ZR, ZR)])
    plsc.subcore_barrier()

    for r in range(NB // NW):
        b = w + NW * r
        pltpu.sync_copy(g_hbm.at[pl.ds(b * BW, BW)], tbl.at[pl.ds(s * BW, BW)])
        pltpu.sync_copy(cntsT_hbm.at[b], crow)

        def outer(w2, carry):
            pltpu.sync_copy(lists_hbm.at[w2, b], recs)
            cv = plsc.load_gather(crow, [jnp.full((16,), w2, jnp.int32)])
            nch = (cv[0] + (CHB - 1)) >> 6

            def chunk(ch, carry2):
                base = ch * CHB
                for grp in range(CHB // 16):
                    rec16 = recs[pl.ds(base + grp * 16, 16)]
                    dstb[0, pl.ds(grp * 16, 16)] = rec16 & 16383
                    lsb[0, pl.ds(grp * 16, 16)] = (rec16 >> 14) + s * BW
                pltpu.sync_copy(tbl.at[lsb.at[0]], stage.at[0])
                pltpu.sync_copy(stage.at[0], acc.at[dstb.at[0]], add=True)
                return carry2

            lax.fori_loop(0, nch, chunk, 0)
            return carry

        lax.fori_loop(0, NW, outer, 0)

    plsc.subcore_barrier()

    @pl.when(s == 0)
    def _():
        pltpu.sync_copy(acc, out_hbm.at[c])


_edge_call = functools.partial(
    pl.kernel,
    out_type=jax.ShapeDtypeStruct((NC, NPAD, D), jnp.float32),
    scratch_types=[
        pltpu.VMEM_SHARED((NS * BW, D), jnp.float32),
        pltpu.VMEM((2, CHB, D), jnp.float32),
        pltpu.VMEM((CAP,), jnp.int32),
        pltpu.VMEM((NW,), jnp.int32),
        pltpu.VMEM((2, CHB), jnp.int32),
        pltpu.VMEM((2, CHB), jnp.int32),
        [pltpu.SemaphoreType.DMA for _ in range(2)],
        [pltpu.SemaphoreType.DMA for _ in range(2)],
        pltpu.VMEM_SHARED((NPAD, D), jnp.float32),
    ],
    compiler_params=pltpu.CompilerParams(needs_layout_passes=False),
    **_MESH,
)(_edge_body)


def _tc_mm_body(x_ref, w_ref, o_ref):
    o_ref[...] = jnp.dot(x_ref[...], w_ref[...],
                         preferred_element_type=jnp.float32)


def _tc_scale_body(h0_ref, degp_ref, o_ref):
    o_ref[...] = h0_ref[...] * _dinv_col(degp_ref[...])


def _tc_combine_body(aggp_ref, g1_ref, degp_ref, b_ref, w2_ref, o_ref):
    dinv = _dinv_col(degp_ref[...])
    tot = aggp_ref[0] + aggp_ref[1] + g1_ref[...]
    h = _elu(dinv * tot + b_ref[...])
    g2 = jnp.dot(h, w2_ref[...], preferred_element_type=jnp.float32) * dinv
    row = lax.broadcasted_iota(jnp.int32, (NPAD, 1), 0)
    o_ref[...] = jnp.where(row < N, g2, 0.0)


def _tc_pool_body(aggp_ref, g2_ref, degp_ref, b_ref, batch_ref, o_ref):
    dinv = _dinv_col(degp_ref[...])
    tot = aggp_ref[0] + aggp_ref[1] + g2_ref[...]
    out2 = _elu(dinv * tot + b_ref[...])
    bcol = batch_ref[...]
    onehot = (bcol == lax.broadcasted_iota(jnp.int32, (1, G), 1)
              ).astype(jnp.float32)
    sums = lax.dot_general(onehot, out2, (((0,), (0,)), ((), ())),
                           preferred_element_type=jnp.float32)
    cnt = jnp.sum(onehot, axis=0).reshape(G, 1)
    o_ref[...] = sums / jnp.maximum(cnt, 1.0)


def _tc(body, out_shape):
    return pl.pallas_call(body, out_shape=out_shape)


def kernel(x, edge_index, batch, W1, b1, Wmu, bmu, Wsig, bsig):
    f32 = jnp.float32
    x_pad = jnp.pad(x, ((0, NPAD - N), (0, 0)))
    src_d = jnp.pad(edge_index[0], (0, EPAD - E),
                    constant_values=N).reshape(NW, RPTD, CH)
    dst_d = jnp.pad(edge_index[1], (0, EPAD - E),
                    constant_values=N).reshape(NW, RPTD, CH)
    batch_col = jnp.pad(batch, (0, NPAD - N), constant_values=G).reshape(NPAD, 1)
    zeros = jnp.zeros((ZR, D), f32)
    W2 = jnp.concatenate([Wmu, Wsig], axis=1)
    b2 = jnp.concatenate([bmu, bsig]).reshape(1, D)
    b1r = b1.reshape(1, D)

    degp, lists, cnts = _bkt_call(src_d, dst_d, zeros)
    degp = degp.reshape(NC, HR * HC, 1)
    cntsT = cnts.T

    h0 = _tc(_tc_mm_body, jax.ShapeDtypeStruct((NPAD, D), f32))(x_pad, W1)
    g1 = _tc(_tc_scale_body, jax.ShapeDtypeStruct((NPAD, D), f32))(h0, degp)

    agg1 = _edge_call(g1, lists, cntsT, zeros)

    g2 = _tc(_tc_combine_body, jax.ShapeDtypeStruct((NPAD, D), f32))(
        agg1, g1, degp, b1r, W2)

    agg2 = _edge_call(g2, lists, cntsT, zeros)

    z = _tc(_tc_pool_body, jax.ShapeDtypeStruct((G, D), f32))(
        agg2, g2, degp, b2, batch_col)

    return (z[:, : D // 2], z[:, D // 2:])


# speedup vs baseline: 1.2382x; 1.0349x over previous
import functools

import jax
import jax.numpy as jnp
from jax import lax
from jax.experimental import pallas as pl
from jax.experimental.pallas import tpu as pltpu
from jax.experimental.pallas import tpu_sc as plsc

N = 10000
E = 320000
D = 128
G = 64
NC = 2
NS = 16
NW = NC * NS
CH = 128
CTP = 2688
EPAD = CTP * CH
EPTD = EPAD // NW
RPTD = EPTD // CH
NPAD = 10240
HR = 80
HC = 128
ZR = NPAD // NS
NB = 64
BW = NPAD // NB
CAP = 384
CHB = 64
DUMMY = N


def _elu(v):
    return jnp.where(v > 0, v, jnp.exp(v) - 1.0)


def _dinv_bcast(degp):
    dinv = lax.rsqrt(degp[0] + degp[1] + 1.0)
    ident = (lax.broadcasted_iota(jnp.int32, (HC, HC), 0)
             == lax.broadcasted_iota(jnp.int32, (HC, HC), 1)
             ).astype(jnp.float32)
    a = (dinv[:, None, :] * ident[None, :, :]).reshape(NPAD, HC)
    return jnp.dot(a, jnp.ones((HC, D), jnp.float32),
                   preferred_element_type=jnp.float32)


_MESH = dict(mesh=plsc.VectorSubcoreMesh(core_axis_name="c",
                                         subcore_axis_name="s"))


def _bkt_body(src_hbm, dst_hbm, z_hbm, deg_hbm, lists_hbm, cnts_hbm,
              srcv, dstv, stg, cnt, hist, rowidx, degacc):
    c = lax.axis_index("c")
    s = lax.axis_index("s")
    w = c * NS + s
    pltpu.sync_copy(src_hbm.at[w], srcv)
    pltpu.sync_copy(dst_hbm.at[w], dstv)
    pltpu.sync_copy(z_hbm.at[pl.ds(0, HR)], hist)
    for i in range(HR // 16):
        rowidx[pl.ds(i * 16, 16)] = lax.iota(jnp.int32, 16) + (16 * i)

    @pl.when(s == 0)
    def _():
        pltpu.sync_copy(z_hbm.at[pl.ds(0, HR)], degacc)

    dm = jnp.full((16,), DUMMY, jnp.int32)

    def fill(r, carry):
        for q in range(CAP // 16):
            stg[r, pl.ds(q * 16, 16)] = dm
        return carry

    lax.fori_loop(0, NB, fill, 0)
    zi = jnp.zeros((16,), jnp.int32)
    for r in range(NB // 16):
        cnt[pl.ds(r * 16, 16)] = zi

    plsc.subcore_barrier()

    ones = jnp.ones((16,), jnp.float32)

    def step(i, carry):
        dv = dstv[i >> 3, pl.ds((i & 7) * 16, 16)]
        plsc.addupdate_scatter(hist, [dv >> 7, dv & 127], ones)
        sv = srcv[i >> 3, pl.ds((i & 7) * 16, 16)]
        b16 = ((sv >> 5) * 52429) >> 18
        rank, lastm = plsc.scan_count(b16)
        base = plsc.load_gather(cnt, [b16])
        pos = jnp.minimum(base + rank - 1, CAP - 1)
        rec = ((sv - b16 * BW) << 14) | dv
        plsc.store_scatter(stg, [b16, pos], rec)
        plsc.store_scatter(cnt, [b16], jnp.minimum(base + rank, CAP),
                           mask=lastm)
        return carry

    lax.fori_loop(0, EPTD // 16, step, 0)

    pltpu.sync_copy(stg, lists_hbm.at[w])
    pltpu.sync_copy(cnt, cnts_hbm.at[w])
    plsc.subcore_barrier()
    pltpu.sync_copy(hist, degacc.at[rowidx], add=True)
    plsc.subcore_barrier()

    @pl.when(s == 0)
    def _():
        pltpu.sync_copy(degacc, deg_hbm.at[c])


_bkt_call = functools.partial(
    pl.kernel,
    out_type=[jax.ShapeDtypeStruct((NC, HR, HC), jnp.float32),
              jax.ShapeDtypeStruct((NW, NB, CAP), jnp.int32),
              jax.ShapeDtypeStruct((NW, NB), jnp.int32)],
    scratch_types=[
        pltpu.VMEM((RPTD, CH), jnp.int32),
        pltpu.VMEM((RPTD, CH), jnp.int32),
        pltpu.VMEM((NB, CAP), jnp.int32),
        pltpu.VMEM((NB,), jnp.int32),
        pltpu.VMEM((HR, HC), jnp.float32),
        pltpu.VMEM((HR,), jnp.int32),
        pltpu.VMEM_SHARED((HR, HC), jnp.float32),
    ],
    compiler_params=pltpu.CompilerParams(needs_layout_passes=False),
    **_MESH,
)(_bkt_body)


def _edge_body(g_hbm, lists_hbm, cntsT_hbm, z_hbm, out_hbm,
               tbl, stage, recs, crow, dstb, lsb, gsems, ssems, acc):
    c = lax.axis_index("c")
    s = lax.axis_index("s")
    w = c * NS + s
    pltpu.sync_copy(z_hbm, acc.at[pl.ds(s * ZR, ZR)])
    plsc.subcore_barrier()

    for r in range(NB // NW):
        b = w + NW * r
        pltpu.sync_copy(g_hbm.at[pl.ds(b * BW, BW)], tbl.at[pl.ds(s * BW, BW)])
        pltpu.sync_copy(cntsT_hbm.at[b], crow)

        def outer(w2, carry):
            pltpu.sync_copy(lists_hbm.at[w2, b], recs)
            cv = plsc.load_gather(crow, [jnp.full((16,), w2, jnp.int32)])
            nch = (cv[0] + (CHB - 1)) >> 6

            def chunk(ch, carry2):
                base = ch * CHB
                for grp in range(CHB // 16):
                    rec16 = recs[pl.ds(base + grp * 16, 16)]
                    dstb[0, pl.ds(grp * 16, 16)] = rec16 & 16383
                    lsb[0, pl.ds(grp * 16, 16)] = (rec16 >> 14) + s * BW
                pltpu.sync_copy(tbl.at[lsb.at[0]], stage.at[0])
                pltpu.sync_copy(stage.at[0], acc.at[dstb.at[0]], add=True)
                return carry2

            lax.fori_loop(0, nch, chunk, 0)
            return carry

        lax.fori_loop(0, NW, outer, 0)

    plsc.subcore_barrier()

    @pl.when(s == 0)
    def _():
        pltpu.sync_copy(acc, out_hbm.at[c])


_edge_call = functools.partial(
    pl.kernel,
    out_type=jax.ShapeDtypeStruct((NC, NPAD, D), jnp.float32),
    scratch_types=[
        pltpu.VMEM_SHARED((NS * BW, D), jnp.float32),
        pltpu.VMEM((2, CHB, D), jnp.float32),
        pltpu.VMEM((CAP,), jnp.int32),
        pltpu.VMEM((NW,), jnp.int32),
        pltpu.VMEM((2, CHB), jnp.int32),
        pltpu.VMEM((2, CHB), jnp.int32),
        [pltpu.SemaphoreType.DMA for _ in range(2)],
        [pltpu.SemaphoreType.DMA for _ in range(2)],
        pltpu.VMEM_SHARED((NPAD, D), jnp.float32),
    ],
    compiler_params=pltpu.CompilerParams(needs_layout_passes=False),
    **_MESH,
)(_edge_body)


def _tc_mm_body(x_ref, w_ref, o_ref):
    o_ref[...] = jnp.dot(x_ref[...], w_ref[...],
                         preferred_element_type=jnp.float32)


def _tc_scale_body(h0_ref, degp_ref, o_ref):
    o_ref[...] = h0_ref[...] * _dinv_bcast(degp_ref[...])


def _tc_combine_body(aggp_ref, g1_ref, degp_ref, b_ref, w2_ref, o_ref):
    dinv = _dinv_bcast(degp_ref[...])
    tot = aggp_ref[0] + aggp_ref[1] + g1_ref[...]
    h = _elu(dinv * tot + b_ref[...])
    g2 = jnp.dot(h, w2_ref[...], preferred_element_type=jnp.float32) * dinv
    row = lax.broadcasted_iota(jnp.int32, (NPAD, 1), 0)
    o_ref[...] = jnp.where(row < N, g2, 0.0)


def _tc_pool_body(aggp_ref, g2_ref, degp_ref, b_ref, batch_ref, o_ref):
    dinv = _dinv_bcast(degp_ref[...])
    tot = aggp_ref[0] + aggp_ref[1] + g2_ref[...]
    out2 = _elu(dinv * tot + b_ref[...])
    bcol = batch_ref[...]
    onehot = (bcol == lax.broadcasted_iota(jnp.int32, (1, G), 1)
              ).astype(jnp.float32)
    sums = lax.dot_general(onehot, out2, (((0,), (0,)), ((), ())),
                           preferred_element_type=jnp.float32)
    cnt = jnp.sum(onehot, axis=0).reshape(G, 1)
    o_ref[...] = sums / jnp.maximum(cnt, 1.0)


def _tc(body, out_shape):
    return pl.pallas_call(body, out_shape=out_shape)


def kernel(x, edge_index, batch, W1, b1, Wmu, bmu, Wsig, bsig):
    f32 = jnp.float32
    x_pad = jnp.pad(x, ((0, NPAD - N), (0, 0)))
    src_d = jnp.pad(edge_index[0], (0, EPAD - E),
                    constant_values=N).reshape(NW, RPTD, CH)
    dst_d = jnp.pad(edge_index[1], (0, EPAD - E),
                    constant_values=N).reshape(NW, RPTD, CH)
    batch_col = jnp.pad(batch, (0, NPAD - N), constant_values=G).reshape(NPAD, 1)
    zeros = jnp.zeros((ZR, D), f32)
    W2 = jnp.concatenate([Wmu, Wsig], axis=1)
    b2 = jnp.concatenate([bmu, bsig]).reshape(1, D)
    b1r = b1.reshape(1, D)

    degp, lists, cnts = _bkt_call(src_d, dst_d, zeros)
    cntsT = cnts.T

    h0 = _tc(_tc_mm_body, jax.ShapeDtypeStruct((NPAD, D), f32))(x_pad, W1)
    g1 = _tc(_tc_scale_body, jax.ShapeDtypeStruct((NPAD, D), f32))(h0, degp)

    agg1 = _edge_call(g1, lists, cntsT, zeros)

    g2 = _tc(_tc_combine_body, jax.ShapeDtypeStruct((NPAD, D), f32))(
        agg1, g1, degp, b1r, W2)

    agg2 = _edge_call(g2, lists, cntsT, zeros)

    z = _tc(_tc_pool_body, jax.ShapeDtypeStruct((G, D), f32))(
        agg2, g2, degp, b2, batch_col)

    return (z[:, : D // 2], z[:, D // 2:])
